# baseline (speedup 1.0000x reference)
import functools

import jax
import jax.numpy as jnp
from jax import lax
from jax.experimental import pallas as pl
from jax.experimental.pallas import tpu as pltpu
from jax.experimental.pallas import tpu_sc as plsc

BATCH = 4096
SEQ = 200
DIM = 64
NUM_CORES = 2
NUM_SUBCORES = 16
NUM_WORKERS = NUM_CORES * NUM_SUBCORES
BPW = BATCH // NUM_WORKERS
CHUNK = 100
NCHUNK = SEQ // CHUNK
LANES = 16
NVEC = DIM // LANES


def _swem_body(idx_hbm, table_hbm, out_hbm, idx_v, rows_a, rows_b, out_v,
               sem_a, sem_b):
    wid = lax.axis_index("s") * NUM_CORES + lax.axis_index("c")
    base = wid * BPW

    pltpu.sync_copy(idx_hbm.at[pl.ds(base, BPW)], idx_v)

    def fire(e, rows_ref, sem):
        for j in range(NCHUNK):
            pltpu.async_copy(
                table_hbm.at[idx_v.at[e, j]],
                rows_ref.at[pl.ds(j * CHUNK, CHUNK)],
                sem,
            )

    def drain(rows_ref, sem):
        for j in range(NCHUNK):
            pltpu.make_async_copy(
                table_hbm.at[idx_v.at[0, j]],
                rows_ref.at[pl.ds(j * CHUNK, CHUNK)],
                sem,
            ).wait()

    def reduce_into(e, rows_ref):
        def body(r, accs):
            return tuple(
                accs[c] + rows_ref[r, pl.ds(c * LANES, LANES)]
                for c in range(NVEC)
            )

        zero = jnp.zeros((LANES,), jnp.float32)
        accs = lax.fori_loop(0, SEQ, body, (zero,) * NVEC, unroll=8)
        for c in range(NVEC):
            out_v[e, pl.ds(c * LANES, LANES)] = accs[c] * (1.0 / SEQ)

    fire(0, rows_a, sem_a)

    def outer(i, _):
        e = 2 * i
        fire(e + 1, rows_b, sem_b)
        drain(rows_a, sem_a)
        reduce_into(e, rows_a)

        @pl.when(e + 2 < BPW)
        def _():
            fire(e + 2, rows_a, sem_a)

        drain(rows_b, sem_b)
        reduce_into(e + 1, rows_b)
        return 0

    lax.fori_loop(0, BPW // 2, outer, 0)

    pltpu.sync_copy(out_v, out_hbm.at[pl.ds(base, BPW)])


@functools.partial(jax.jit, static_argnums=())
def _swem_sc(idx, table):
    mesh = plsc.VectorSubcoreMesh(
        core_axis_name="c",
        subcore_axis_name="s",
        num_cores=NUM_CORES,
        num_subcores=NUM_SUBCORES,
    )
    k = pl.kernel(
        _swem_body,
        out_type=jax.ShapeDtypeStruct((BATCH, DIM), jnp.float32),
        mesh=mesh,
        scratch_types=[
            pltpu.VMEM((BPW, NCHUNK, CHUNK), jnp.int32),
            pltpu.VMEM((SEQ, DIM), jnp.float32),
            pltpu.VMEM((SEQ, DIM), jnp.float32),
            pltpu.VMEM((BPW, DIM), jnp.float32),
            pltpu.SemaphoreType.DMA,
            pltpu.SemaphoreType.DMA,
        ],
        compiler_params=pltpu.CompilerParams(use_tc_tiling_on_sc=False),
    )
    return k(idx, table)


def kernel(input, table):
    idx = input.reshape(BATCH, NCHUNK, CHUNK)
    return _swem_sc(idx, table)
